# Initial kernel scaffold; baseline (speedup 1.0000x reference)
import jax
import jax.numpy as jnp
from jax.experimental import pallas as pl


def kernel(x, edge_index, W_in, b_in, W0, b0, g0, beta0, W1, b1, g1, beta1, W2, b2, g2, beta2, Wh1, bh1, Wh2, bh2):
    raise NotImplementedError("write your pallas kernel here")


# speedup vs baseline: 6.5487x; 6.5487x over previous
import functools

import jax
import jax.numpy as jnp
from jax import lax
from jax.experimental import pallas as pl
from jax.experimental.pallas import tpu as pltpu
from jax.experimental.pallas import tpu_sc as plsc

N = 10000
E = 160000
IN_DIM = 128
H = 256
HH = H // 2
EPS = 1e-5

NCORE = 2
NSUB = 16
RPT = 624
REM_BASE = RPT * NSUB
REM_ROWS = N - REM_BASE
ZROWS = 16

AGG_CH = 80
AGG_EDGES_PER_TILE = E // NSUB
AGG_ITERS = AGG_EDGES_PER_TILE // AGG_CH

DEG_CH = 40
DEG_EDGES_PER_TILE = E // (NSUB * NCORE)
DEG_ITERS = DEG_EDGES_PER_TILE // DEG_CH

MXU_PREC = lax.Precision.HIGHEST


def _fill(buf, rows, width, vec):
    for j in range(rows):
        for k in range(width // 16):
            buf[j, pl.ds(k * 16, 16)] = vec


def _deg_body(dst_hbm, out_hbm, didx, ones, zbuf, dacc):
    c = lax.axis_index("c")
    s = lax.axis_index("s")
    _fill(ones, DEG_CH, HH, jnp.ones((16,), jnp.float32))
    _fill(zbuf, ZROWS, HH, jnp.zeros((16,), jnp.float32))
    for k in range(RPT // ZROWS):
        pltpu.sync_copy(zbuf, dacc.at[pl.ds(s * RPT + k * ZROWS, ZROWS)])

    @pl.when(s == NSUB - 1)
    def _():
        pltpu.sync_copy(zbuf, dacc.at[pl.ds(REM_BASE, REM_ROWS)])

    plsc.subcore_barrier()
    ebase = (c * NSUB + s) * DEG_EDGES_PER_TILE

    def body(i, carry):
        b = ebase + i * DEG_CH
        pltpu.sync_copy(dst_hbm.at[pl.ds(b, DEG_CH)], didx)
        pltpu.sync_copy(ones, dacc.at[didx], add=True)
        return carry

    lax.fori_loop(0, DEG_ITERS, body, 0)
    plsc.subcore_barrier()
    pltpu.sync_copy(dacc.at[pl.ds(s * RPT, RPT)],
                    out_hbm.at[pl.ds(c * N + s * RPT, RPT)])

    @pl.when(s == NSUB - 1)
    def _():
        pltpu.sync_copy(dacc.at[pl.ds(REM_BASE, REM_ROWS)],
                        out_hbm.at[pl.ds(c * N + REM_BASE, REM_ROWS)])


def _agg_body(y_hbm, src_hbm, dst_hbm, out_hbm,
              sidx, didx, gidx, rows, zbuf, acc, sem):
    c = lax.axis_index("c")
    s = lax.axis_index("s")
    _fill(zbuf, ZROWS, HH, jnp.zeros((16,), jnp.float32))
    for k in range(RPT // ZROWS):
        pltpu.sync_copy(zbuf, acc.at[pl.ds(s * RPT + k * ZROWS, ZROWS)])

    @pl.when(s == NSUB - 1)
    def _():
        pltpu.sync_copy(zbuf, acc.at[pl.ds(REM_BASE, REM_ROWS)])

    plsc.subcore_barrier()
    ebase = s * AGG_EDGES_PER_TILE
    off = c * N

    def body(i, carry):
        b = ebase + i * AGG_CH
        pltpu.sync_copy(src_hbm.at[pl.ds(b, AGG_CH)], sidx)
        pltpu.sync_copy(dst_hbm.at[pl.ds(b, AGG_CH)], didx)
        for k in range(AGG_CH // 16):
            gidx[pl.ds(k * 16, 16)] = sidx[pl.ds(k * 16, 16)] + off
        pltpu.async_copy(y_hbm.at[gidx], rows, sem).wait()
        pltpu.sync_copy(rows, acc.at[didx], add=True)
        return carry

    lax.fori_loop(0, AGG_ITERS, body, 0)
    plsc.subcore_barrier()
    pltpu.sync_copy(acc.at[pl.ds(s * RPT, RPT)],
                    out_hbm.at[pl.ds(c * N + s * RPT, RPT)])

    @pl.when(s == NSUB - 1)
    def _():
        pltpu.sync_copy(acc.at[pl.ds(REM_BASE, REM_ROWS)],
                        out_hbm.at[pl.ds(c * N + REM_BASE, REM_ROWS)])


@functools.cache
def _sc_kernels():
    mesh = plsc.VectorSubcoreMesh(
        core_axis_name="c", subcore_axis_name="s",
        num_cores=NCORE, num_subcores=NSUB)
    deg = pl.kernel(
        _deg_body,
        out_type=jax.ShapeDtypeStruct((NCORE * N, HH), jnp.float32),
        mesh=mesh,
        scratch_types=[
            pltpu.VMEM((DEG_CH,), jnp.int32),
            pltpu.VMEM((DEG_CH, HH), jnp.float32),
            pltpu.VMEM((ZROWS, HH), jnp.float32),
            pltpu.VMEM_SHARED((N, HH), jnp.float32),
        ],
    )
    agg = pl.kernel(
        _agg_body,
        out_type=jax.ShapeDtypeStruct((NCORE * N, HH), jnp.float32),
        mesh=mesh,
        scratch_types=[
            pltpu.VMEM((AGG_CH,), jnp.int32),
            pltpu.VMEM((AGG_CH,), jnp.int32),
            pltpu.VMEM((AGG_CH,), jnp.int32),
            pltpu.VMEM((AGG_CH, HH), jnp.float32),
            pltpu.VMEM((ZROWS, HH), jnp.float32),
            pltpu.VMEM_SHARED((N, HH), jnp.float32),
            pltpu.SemaphoreType.DMA,
        ],
    )
    return deg, agg


BLK = 1000
GRID = N // BLK


def _dinv(p0_ref, p1_ref):
    d = 1.0 + p0_ref[:, 0:1] + p1_ref[:, 0:1]
    return lax.rsqrt(d)


def _pre_body(x_ref, win_ref, bin_ref, w0_ref, p0_ref, p1_ref, h_ref, y_ref):
    h = jnp.dot(x_ref[...], win_ref[...], preferred_element_type=jnp.float32,
                precision=MXU_PREC) + bin_ref[...]
    h_ref[...] = h
    dinv = _dinv(p0_ref, p1_ref)
    y = dinv * jnp.dot(h, w0_ref[...], preferred_element_type=jnp.float32,
                       precision=MXU_PREC)
    y_ref[0] = y[:, :HH]
    y_ref[1] = y[:, HH:]


def _update(h_ref, y_ref, a_ref, dinv, b_ref, g_ref, be_ref):
    aggy = jnp.concatenate([a_ref[0] + y_ref[0], a_ref[1] + y_ref[1]], axis=1)
    u = h_ref[...] + dinv * aggy + b_ref[...]
    m = jnp.mean(u, axis=1, keepdims=True)
    v = jnp.mean((u - m) ** 2, axis=1, keepdims=True)
    hn = (u - m) * lax.rsqrt(v + EPS) * g_ref[...] + be_ref[...]
    return jnp.maximum(hn, 0.0)


def _mid_body(h_ref, y_ref, a_ref, p0_ref, p1_ref, b_ref, g_ref, be_ref,
              wn_ref, ho_ref, yo_ref):
    dinv = _dinv(p0_ref, p1_ref)
    h = _update(h_ref, y_ref, a_ref, dinv, b_ref, g_ref, be_ref)
    ho_ref[...] = h
    y = dinv * jnp.dot(h, wn_ref[...], preferred_element_type=jnp.float32,
                       precision=MXU_PREC)
    yo_ref[0] = y[:, :HH]
    yo_ref[1] = y[:, HH:]


def _post_body(h_ref, y_ref, a_ref, p0_ref, p1_ref, b_ref, g_ref, be_ref,
               wh1_ref, bh1_ref, wh2_ref, bh2_ref, o_ref):
    dinv = _dinv(p0_ref, p1_ref)
    h = _update(h_ref, y_ref, a_ref, dinv, b_ref, g_ref, be_ref)
    t = jnp.maximum(jnp.dot(h, wh1_ref[...], preferred_element_type=jnp.float32,
                            precision=MXU_PREC) + bh1_ref[...], 0.0)
    o = jnp.dot(t, wh2_ref[...], preferred_element_type=jnp.float32,
                precision=MXU_PREC) + bh2_ref[...]
    o_ref[...] = 1.0 / (1.0 + jnp.exp(-o))


def _row_spec(w):
    return pl.BlockSpec((BLK, w), lambda i: (i, 0))


def _full_spec(shape):
    nd = len(shape)
    return pl.BlockSpec(shape, lambda i, _nd=nd: (0,) * _nd)


def _y_spec():
    return pl.BlockSpec((NCORE, BLK, HH), lambda i: (0, i, 0))


_pre_call = pl.pallas_call(
    _pre_body,
    grid=(GRID,),
    in_specs=[_row_spec(IN_DIM), _full_spec((IN_DIM, H)), _full_spec((1, H)),
              _full_spec((H, H)), _row_spec(HH), _row_spec(HH)],
    out_specs=[_row_spec(H), _y_spec()],
    out_shape=[jax.ShapeDtypeStruct((N, H), jnp.float32),
               jax.ShapeDtypeStruct((NCORE, N, HH), jnp.float32)],
)

_mid_call = pl.pallas_call(
    _mid_body,
    grid=(GRID,),
    in_specs=[_row_spec(H), _y_spec(), _y_spec(), _row_spec(HH), _row_spec(HH),
              _full_spec((1, H)), _full_spec((1, H)), _full_spec((1, H)),
              _full_spec((H, H))],
    out_specs=[_row_spec(H), _y_spec()],
    out_shape=[jax.ShapeDtypeStruct((N, H), jnp.float32),
               jax.ShapeDtypeStruct((NCORE, N, HH), jnp.float32)],
)

_post_call = pl.pallas_call(
    _post_body,
    grid=(GRID,),
    in_specs=[_row_spec(H), _y_spec(), _y_spec(), _row_spec(HH), _row_spec(HH),
              _full_spec((1, H)), _full_spec((1, H)), _full_spec((1, H)),
              _full_spec((H, HH)), _full_spec((1, HH)),
              _full_spec((HH, 1)), _full_spec((1, 1))],
    out_specs=[_row_spec(1)],
    out_shape=[jax.ShapeDtypeStruct((N, 1), jnp.float32)],
)


def kernel(x, edge_index, W_in, b_in, W0, b0, g0, beta0, W1, b1, g1, beta1,
           W2, b2, g2, beta2, Wh1, bh1, Wh2, bh2):
    src = edge_index[0]
    dst = edge_index[1]
    _deg_kernel, _agg_kernel = _sc_kernels()

    degp = _deg_kernel(dst)
    p0 = degp[:N]
    p1 = degp[N:]

    r1 = lambda a: a.reshape(1, -1)
    h0, y0 = _pre_call(x, W_in, r1(b_in), W0, p0, p1)
    a0 = _agg_kernel(y0.reshape(NCORE * N, HH), src, dst).reshape(NCORE, N, HH)
    h1, y1 = _mid_call(h0, y0, a0, p0, p1, r1(b0), r1(g0), r1(beta0), W1)
    a1 = _agg_kernel(y1.reshape(NCORE * N, HH), src, dst).reshape(NCORE, N, HH)
    h2, y2 = _mid_call(h1, y1, a1, p0, p1, r1(b1), r1(g1), r1(beta1), W2)
    a2 = _agg_kernel(y2.reshape(NCORE * N, HH), src, dst).reshape(NCORE, N, HH)
    (out,) = _post_call(h2, y2, a2, p0, p1, r1(b2), r1(g2), r1(beta2),
                        Wh1, r1(bh1), Wh2, bh2.reshape(1, 1))
    return out[:, 0]
